# baseline (speedup 1.0000x reference)
import functools

import jax
import jax.numpy as jnp
import numpy as np
from jax import lax
from jax.experimental import pallas as pl
from jax.experimental.pallas import tpu as pltpu
from jax.experimental.pallas import tpu_sc as plsc

_B = 16384
_NW = 32
_CHUNK = _B // _NW
_STEPS = _CHUNK // 16
_BIG = 2**31 - 1

_OV1 = 0
_OV2 = 2 * _CHUNK
_OV3 = 4 * _CHUNK
_OXY = 6 * _CHUNK
_OYZ = 8 * _CHUNK
_OXZ = 10 * _CHUNK
_OFL = 12 * _CHUNK
_ROW = 13 * _CHUNK

_POS = [(0, 4, 4), (0, 6, 4), (1, 5, 5), (1, 6, 5), (2, 4, 4), (2, 5, 5),
        (2, 6, 6), (2, 7, 7), (4, 0, 4), (4, 2, 4), (5, 1, 5), (5, 2, 5),
        (6, 2, 6), (7, 2, 7)]
_NEG = [(0, 4, 1), (0, 4, 2), (0, 6, 1), (0, 6, 2), (1, 5, 0), (1, 5, 2),
        (1, 6, 0), (1, 6, 2), (2, 4, 1), (2, 4, 2), (2, 5, 0), (2, 5, 2),
        (4, 0, 1), (4, 0, 2), (4, 2, 1), (4, 2, 2), (5, 1, 0), (5, 1, 2),
        (5, 2, 0), (5, 2, 2), (2, 7, 2), (7, 2, 2)]
_NR = len(_NEG)
assert _NR <= 32


def _dm(cat):
    return 0 if cat < 4 else 1


def _code(t):
    return t[0] * 64 + t[1] * 8 + t[2]


_POS_GROUPS = {}
for _t in _POS:
    _key = (_dm(_t[0]), _dm(_t[1]), _dm(_t[2]))
    _POS_GROUPS.setdefault(_key, []).append(_code(_t))

_CONST_NP = np.zeros((8, 32), np.float32)
for _r, _t in enumerate(_NEG):
    _CONST_NP[0, _r] = float(_dm(_t[0]))
    _CONST_NP[1, _r] = float(_dm(_t[1]))
    _CONST_NP[2, _r] = float(_dm(_t[2]))


def _log1mexp(x):
    return jnp.log(1.0 - jnp.exp(x))


@functools.cache
def _build_sc_scan():
  mesh = plsc.VectorSubcoreMesh(core_axis_name="c", subcore_axis_name="s")

  @functools.partial(
    pl.kernel,
    mesh=mesh,
    compiler_params=pltpu.CompilerParams(needs_layout_passes=False),
    out_type=jax.ShapeDtypeStruct((_NW, 23, 16), jnp.float32),
    scratch_types=[
        pltpu.VMEM((_ROW,), jnp.float32),
        pltpu.VMEM((_CHUNK,), jnp.int32),
        pltpu.VMEM((23, 16), jnp.float32),
        pltpu.VMEM_SHARED((16, _ROW), jnp.float32),
        pltpu.VMEM_SHARED((16, 23, 16), jnp.float32),
        pltpu.SemaphoreType.DMA,
    ],
  )
  def _sc_scan(pk_hbm, out_hbm, pk, codec, stg, shin, shout, sem):
    cid = lax.axis_index("c")
    sid = lax.axis_index("s")
    wid = cid * 16 + sid

    @pl.when(sid == 0)
    def _():
        pltpu.sync_copy(pk_hbm.at[pl.ds(cid * 16, 16)], shin)

    plsc.subcore_barrier()
    pltpu.sync_copy(shin.at[sid], pk)

    lane = lax.broadcasted_iota(jnp.int32, (16,), 0)

    def igather(off, idxv):
        return plsc.bitcast(plsc.load_gather(pk, [idxv + off]), jnp.int32)

    def p1_body(i, pos_acc):
        e = lane * 2 + i * 32
        o = e + 1
        x0 = igather(_OXY, e)
        x1 = igather(_OXY, o)
        y0 = igather(_OYZ, e)
        y1 = igather(_OYZ, o)
        z0 = igather(_OXZ, e)
        z1 = igather(_OXZ, o)
        fl4 = 4 * plsc.bitcast(pk[pl.ds(_OFL + i * 16, 16)], jnp.int32)
        cx = 3 - 3 * x0 - 2 * x1 + 4 * x0 * x1 + fl4
        cy = 3 - 3 * y0 - 2 * y1 + 4 * y0 * y1 + fl4
        cz = 3 - 3 * z0 - 2 * z1 + 4 * z0 * z1 + fl4
        code = cx * 64 + cy * 8 + cz
        codec[pl.ds(i * 16, 16)] = code
        cols = {}

        def col(off, c):
            if (off, c) not in cols:
                cols[(off, c)] = plsc.load_gather(pk, [(o if c else e) + off])
            return cols[(off, c)]

        for (f1, f2, f3), codes in sorted(_POS_GROUPS.items()):
            w = col(_OV1, f1) + col(_OV2, f2) - col(_OV3, f3)
            sel = functools.reduce(jnp.logical_or, [code == t for t in codes])
            pos_acc = pos_acc + jnp.where(sel, w, 0.0)
        return pos_acc

    pos_acc = lax.fori_loop(0, _STEPS, p1_body, jnp.zeros((16,), jnp.float32))

    big16 = jnp.full((16,), _BIG, jnp.int32)
    zero16 = jnp.zeros((16,), jnp.int32)

    def scan_pair(ta, tb):
        def body(i, carry):
            m1a, m2a, ca, m1b, m2b, cb = carry
            code = codec[pl.ds(i * 16, 16)]
            idxv = lane + i * 16
            sa = code == ta
            ca = ca + sa.astype(jnp.int32)
            mia = jnp.where(sa, idxv, _BIG)
            m2a = jnp.minimum(m2a, jnp.maximum(m1a, mia))
            m1a = jnp.minimum(m1a, mia)
            sb = code == tb
            cb = cb + sb.astype(jnp.int32)
            mib = jnp.where(sb, idxv, _BIG)
            m2b = jnp.minimum(m2b, jnp.maximum(m1b, mib))
            m1b = jnp.minimum(m1b, mib)
            return (m1a, m2a, ca, m1b, m2b, cb)

        init = (big16, big16, zero16, big16, big16, zero16)
        return lax.fori_loop(0, _STEPS, body, init)

    base = wid * _CHUNK
    cnt_v = [zero16, zero16]
    fg_v = [big16, big16]
    sg_v = [big16, big16]
    fl_v = [zero16, zero16]
    sl_v = [zero16, zero16]

    for r0 in range(0, _NR, 2):
        ta = _code(_NEG[r0])
        tb = _code(_NEG[r0 + 1]) if r0 + 1 < _NR else -1
        m1a, m2a, ca, m1b, m2b, cb = scan_pair(ta, tb)
        recs = [(r0, m1a, m2a, ca)]
        if r0 + 1 < _NR:
            recs.append((r0 + 1, m1b, m2b, cb))
        for r, m1, m2, cv in recs:
            first = jnp.min(m1)
            second = jnp.min(jnp.where(m1 == first, m2, m1))
            cnt = jnp.sum(cv)
            g = r // 16
            sel = lane == (r % 16)
            cnt_v[g] = jnp.where(sel, cnt, cnt_v[g])
            fg_v[g] = jnp.where(sel, jnp.where(first == _BIG, _BIG, first + base), fg_v[g])
            sg_v[g] = jnp.where(sel, jnp.where(second == _BIG, _BIG, second + base), sg_v[g])
            fl_v[g] = jnp.where(sel, jnp.minimum(first, _CHUNK - 1), fl_v[g])
            sl_v[g] = jnp.where(sel, jnp.minimum(second, _CHUNK - 1), sl_v[g])

    def gpair(off, idxv):
        e = idxv * 2 + off
        return plsc.load_gather(pk, [e]) + plsc.load_gather(pk, [e + 1])

    def g1v(off, idxv, c):
        return plsc.load_gather(pk, [idxv * 2 + c + off])

    stg[0, :] = pos_acc
    stg[1, :] = gpair(_OV1, fl_v[0])
    stg[2, :] = gpair(_OV1, fl_v[1])
    stg[3, :] = gpair(_OV1, sl_v[0])
    stg[4, :] = gpair(_OV1, sl_v[1])
    stg[5, :] = gpair(_OV2, fl_v[0])
    stg[6, :] = gpair(_OV2, fl_v[1])
    stg[7, :] = gpair(_OV2, sl_v[0])
    stg[8, :] = gpair(_OV2, sl_v[1])
    stg[9, :] = g1v(_OV3, fl_v[0], 0)
    stg[10, :] = g1v(_OV3, fl_v[1], 0)
    stg[11, :] = g1v(_OV3, sl_v[0], 0)
    stg[12, :] = g1v(_OV3, sl_v[1], 0)
    stg[13, :] = g1v(_OV3, fl_v[0], 1)
    stg[14, :] = g1v(_OV3, fl_v[1], 1)
    stg[15, :] = g1v(_OV3, sl_v[0], 1)
    stg[16, :] = g1v(_OV3, sl_v[1], 1)
    stg[17, :] = plsc.bitcast(cnt_v[0], jnp.float32)
    stg[18, :] = plsc.bitcast(cnt_v[1], jnp.float32)
    stg[19, :] = plsc.bitcast(fg_v[0], jnp.float32)
    stg[20, :] = plsc.bitcast(fg_v[1], jnp.float32)
    stg[21, :] = plsc.bitcast(sg_v[0], jnp.float32)
    stg[22, :] = plsc.bitcast(sg_v[1], jnp.float32)

    pltpu.sync_copy(stg, shout.at[sid])
    plsc.subcore_barrier()

    @pl.when(sid == 0)
    def _():
        pltpu.sync_copy(shout, out_hbm.at[pl.ds(cid * 16, 16)])

  return _sc_scan


def _combine_body(flts_ref, consts_ref, out_ref):
    flts = flts_ref[...]
    f1sel = consts_ref[0:1, :]
    f2sel = consts_ref[1:2, :]
    f3sel = consts_ref[2:3, :]

    ints = lax.bitcast_convert_type(flts[:, 272:368], jnp.int32)
    cnt = ints[:, 0:32]
    first = ints[:, 32:64]
    second = ints[:, 64:96]
    gcnt = jnp.sum(cnt, axis=0, keepdims=True)
    g1 = jnp.min(first, axis=0, keepdims=True)
    g2 = jnp.min(jnp.where(first == g1, second, first), axis=0, keepdims=True)
    p1 = jnp.where(gcnt >= 2, g2, g1)

    s1f = flts[:, 16:48]
    s1s = flts[:, 48:80]
    s2f = flts[:, 80:112]
    s2s = flts[:, 112:144]
    v30f = flts[:, 144:176]
    v30s = flts[:, 176:208]
    v31f = flts[:, 208:240]
    v31s = flts[:, 240:272]

    own0 = first == g1

    def at_p0(q):
        return jnp.sum(jnp.where(own0, q, 0.0), axis=0, keepdims=True)

    def at_p1(qf, qs):
        return (jnp.sum(jnp.where(first == p1, qf, 0.0), axis=0, keepdims=True)
                + jnp.sum(jnp.where(second == p1, qs, 0.0), axis=0, keepdims=True))

    s1_i1 = jnp.where(f1sel > 0.5, at_p1(s1f, s1s), at_p0(s1f))
    s2_i2 = jnp.where(f2sel > 0.5, at_p1(s2f, s2s), at_p0(s2f))
    v30_i3 = jnp.where(f3sel > 0.5, at_p1(v30f, v30s), at_p0(v30f))
    v31_i3 = jnp.where(f3sel > 0.5, at_p1(v31f, v31s), at_p0(v31f))

    lsum = s1_i1 + s2_i2 - (_log1mexp(v30_i3) + _log1mexp(v31_i3))
    neg_total = jnp.sum(jnp.where(gcnt > 0, -lsum, 0.0))
    pos_total = jnp.sum(flts[:, 0:16])
    out_ref[...] = jnp.broadcast_to(neg_total - pos_total, (1, 1))


def kernel(volume1, volume2, volume3, xy_rel_id, yz_rel_id, xz_rel_id, flag):
    f32 = jnp.float32
    i32 = jnp.int32
    parts = [
        volume1.reshape(_NW, 2 * _CHUNK),
        volume2.reshape(_NW, 2 * _CHUNK),
        volume3.reshape(_NW, 2 * _CHUNK),
        lax.bitcast_convert_type(xy_rel_id.astype(i32).reshape(_NW, 2 * _CHUNK), f32),
        lax.bitcast_convert_type(yz_rel_id.astype(i32).reshape(_NW, 2 * _CHUNK), f32),
        lax.bitcast_convert_type(xz_rel_id.astype(i32).reshape(_NW, 2 * _CHUNK), f32),
        lax.bitcast_convert_type(flag.astype(i32).reshape(_NW, _CHUNK), f32),
    ]
    pk = jnp.concatenate(parts, axis=1)
    out_flt = _build_sc_scan()(pk)
    out = pl.pallas_call(
        _combine_body,
        out_shape=jax.ShapeDtypeStruct((1, 1), jnp.float32),
    )(out_flt.reshape(_NW, 23 * 16), jnp.asarray(_CONST_NP))
    return out[0, 0]


# speedup vs baseline: 1.0043x; 1.0043x over previous
import functools

import jax
import jax.numpy as jnp
import numpy as np
from jax import lax
from jax.experimental import pallas as pl
from jax.experimental.pallas import tpu as pltpu
from jax.experimental.pallas import tpu_sc as plsc

_B = 16384
_NW = 32
_CHUNK = _B // _NW
_STEPS = _CHUNK // 16
_BIG = 2**31 - 1

_OV1 = 0
_OV2 = 2 * _CHUNK
_OV3 = 4 * _CHUNK
_OXY = 6 * _CHUNK
_OYZ = 8 * _CHUNK
_OXZ = 10 * _CHUNK
_OFL = 12 * _CHUNK
_ROW = 8192

_POS = [(0, 4, 4), (0, 6, 4), (1, 5, 5), (1, 6, 5), (2, 4, 4), (2, 5, 5),
        (2, 6, 6), (2, 7, 7), (4, 0, 4), (4, 2, 4), (5, 1, 5), (5, 2, 5),
        (6, 2, 6), (7, 2, 7)]
_NEG = [(0, 4, 1), (0, 4, 2), (0, 6, 1), (0, 6, 2), (1, 5, 0), (1, 5, 2),
        (1, 6, 0), (1, 6, 2), (2, 4, 1), (2, 4, 2), (2, 5, 0), (2, 5, 2),
        (4, 0, 1), (4, 0, 2), (4, 2, 1), (4, 2, 2), (5, 1, 0), (5, 1, 2),
        (5, 2, 0), (5, 2, 2), (2, 7, 2), (7, 2, 2)]
_NR = len(_NEG)
assert _NR <= 32


def _dm(cat):
    return 0 if cat < 4 else 1


def _code(t):
    return t[0] * 64 + t[1] * 8 + t[2]


_POS_GROUPS = {}
for _t in _POS:
    _key = (_dm(_t[0]), _dm(_t[1]), _dm(_t[2]))
    _POS_GROUPS.setdefault(_key, []).append(_code(_t))

_CONST_NP = np.zeros((8, 32), np.float32)
for _r, _t in enumerate(_NEG):
    _CONST_NP[0, _r] = float(_dm(_t[0]))
    _CONST_NP[1, _r] = float(_dm(_t[1]))
    _CONST_NP[2, _r] = float(_dm(_t[2]))


def _log1mexp(x):
    return jnp.log(1.0 - jnp.exp(x))


@functools.cache
def _build_sc_scan():
  mesh = plsc.VectorSubcoreMesh(core_axis_name="c", subcore_axis_name="s")

  @functools.partial(
    pl.kernel,
    mesh=mesh,
    compiler_params=pltpu.CompilerParams(needs_layout_passes=False),
    out_type=jax.ShapeDtypeStruct((_NW, 23, 16), jnp.float32),
    scratch_types=[
        pltpu.VMEM((64, 128), jnp.float32),
        pltpu.VMEM((_CHUNK,), jnp.int32),
        pltpu.VMEM((23, 16), jnp.float32),
        pltpu.SemaphoreType.DMA,
    ],
  )
  def _sc_scan(pk_hbm, out_hbm, pk, codec, stg, sem):
    cid = lax.axis_index("c")
    sid = lax.axis_index("s")
    wid = cid * 16 + sid

    lane = lax.broadcasted_iota(jnp.int32, (16,), 0)

    cps = [pltpu.async_copy(pk_hbm.at[wid * 64 + k * 16 + lane],
                            pk.at[pl.ds(k * 16, 16)], sem)
           for k in range(4)]
    for cp in cps:
        cp.wait()

    def fgather(flat):
        return plsc.load_gather(pk, [flat >> 7, flat & 127])

    def igather(off, idxv):
        return plsc.bitcast(fgather(idxv + off), jnp.int32)

    def p1_body(i, pos_acc):
        e = lane * 2 + i * 32
        o = e + 1
        x0 = igather(_OXY, e)
        x1 = igather(_OXY, o)
        y0 = igather(_OYZ, e)
        y1 = igather(_OYZ, o)
        z0 = igather(_OXZ, e)
        z1 = igather(_OXZ, o)
        fl4 = 4 * igather(_OFL, lane + i * 16)
        cx = 3 - 3 * x0 - 2 * x1 + 4 * x0 * x1 + fl4
        cy = 3 - 3 * y0 - 2 * y1 + 4 * y0 * y1 + fl4
        cz = 3 - 3 * z0 - 2 * z1 + 4 * z0 * z1 + fl4
        code = cx * 64 + cy * 8 + cz
        codec[pl.ds(i * 16, 16)] = code
        cols = {}

        def col(off, c):
            if (off, c) not in cols:
                cols[(off, c)] = fgather((o if c else e) + off)
            return cols[(off, c)]

        for (f1, f2, f3), codes in sorted(_POS_GROUPS.items()):
            w = col(_OV1, f1) + col(_OV2, f2) - col(_OV3, f3)
            sel = functools.reduce(jnp.logical_or, [code == t for t in codes])
            pos_acc = pos_acc + jnp.where(sel, w, 0.0)
        return pos_acc

    pos_acc = lax.fori_loop(0, _STEPS, p1_body, jnp.zeros((16,), jnp.float32))

    big16 = jnp.full((16,), _BIG, jnp.int32)
    zero16 = jnp.zeros((16,), jnp.int32)

    def scan_pair(ta, tb):
        def body(i, carry):
            m1a, m2a, ca, m1b, m2b, cb = carry
            code = codec[pl.ds(i * 16, 16)]
            idxv = lane + i * 16
            sa = code == ta
            ca = ca + sa.astype(jnp.int32)
            mia = jnp.where(sa, idxv, _BIG)
            m2a = jnp.minimum(m2a, jnp.maximum(m1a, mia))
            m1a = jnp.minimum(m1a, mia)
            sb = code == tb
            cb = cb + sb.astype(jnp.int32)
            mib = jnp.where(sb, idxv, _BIG)
            m2b = jnp.minimum(m2b, jnp.maximum(m1b, mib))
            m1b = jnp.minimum(m1b, mib)
            return (m1a, m2a, ca, m1b, m2b, cb)

        init = (big16, big16, zero16, big16, big16, zero16)
        return lax.fori_loop(0, _STEPS, body, init)

    base = wid * _CHUNK
    cnt_v = [zero16, zero16]
    fg_v = [big16, big16]
    sg_v = [big16, big16]
    fl_v = [zero16, zero16]
    sl_v = [zero16, zero16]

    for r0 in range(0, _NR, 2):
        ta = _code(_NEG[r0])
        tb = _code(_NEG[r0 + 1]) if r0 + 1 < _NR else -1
        m1a, m2a, ca, m1b, m2b, cb = scan_pair(ta, tb)
        recs = [(r0, m1a, m2a, ca)]
        if r0 + 1 < _NR:
            recs.append((r0 + 1, m1b, m2b, cb))
        for r, m1, m2, cv in recs:
            first = jnp.min(m1)
            second = jnp.min(jnp.where(m1 == first, m2, m1))
            cnt = jnp.sum(cv)
            g = r // 16
            sel = lane == (r % 16)
            cnt_v[g] = jnp.where(sel, cnt, cnt_v[g])
            fg_v[g] = jnp.where(sel, jnp.where(first == _BIG, _BIG, first + base), fg_v[g])
            sg_v[g] = jnp.where(sel, jnp.where(second == _BIG, _BIG, second + base), sg_v[g])
            fl_v[g] = jnp.where(sel, jnp.minimum(first, _CHUNK - 1), fl_v[g])
            sl_v[g] = jnp.where(sel, jnp.minimum(second, _CHUNK - 1), sl_v[g])

    def gpair(off, idxv):
        e = idxv * 2 + off
        return fgather(e) + fgather(e + 1)

    def g1v(off, idxv, c):
        return fgather(idxv * 2 + c + off)

    stg[0, :] = pos_acc
    stg[1, :] = gpair(_OV1, fl_v[0])
    stg[2, :] = gpair(_OV1, fl_v[1])
    stg[3, :] = gpair(_OV1, sl_v[0])
    stg[4, :] = gpair(_OV1, sl_v[1])
    stg[5, :] = gpair(_OV2, fl_v[0])
    stg[6, :] = gpair(_OV2, fl_v[1])
    stg[7, :] = gpair(_OV2, sl_v[0])
    stg[8, :] = gpair(_OV2, sl_v[1])
    stg[9, :] = g1v(_OV3, fl_v[0], 0)
    stg[10, :] = g1v(_OV3, fl_v[1], 0)
    stg[11, :] = g1v(_OV3, sl_v[0], 0)
    stg[12, :] = g1v(_OV3, sl_v[1], 0)
    stg[13, :] = g1v(_OV3, fl_v[0], 1)
    stg[14, :] = g1v(_OV3, fl_v[1], 1)
    stg[15, :] = g1v(_OV3, sl_v[0], 1)
    stg[16, :] = g1v(_OV3, sl_v[1], 1)
    stg[17, :] = plsc.bitcast(cnt_v[0], jnp.float32)
    stg[18, :] = plsc.bitcast(cnt_v[1], jnp.float32)
    stg[19, :] = plsc.bitcast(fg_v[0], jnp.float32)
    stg[20, :] = plsc.bitcast(fg_v[1], jnp.float32)
    stg[21, :] = plsc.bitcast(sg_v[0], jnp.float32)
    stg[22, :] = plsc.bitcast(sg_v[1], jnp.float32)

    pltpu.sync_copy(stg, out_hbm.at[wid])

  return _sc_scan


def _combine_body(flts_ref, consts_ref, out_ref):
    flts = flts_ref[...]
    f1sel = consts_ref[0:1, :]
    f2sel = consts_ref[1:2, :]
    f3sel = consts_ref[2:3, :]

    ints = lax.bitcast_convert_type(flts[:, 272:368], jnp.int32)
    cnt = ints[:, 0:32]
    first = ints[:, 32:64]
    second = ints[:, 64:96]
    gcnt = jnp.sum(cnt, axis=0, keepdims=True)
    g1 = jnp.min(first, axis=0, keepdims=True)
    g2 = jnp.min(jnp.where(first == g1, second, first), axis=0, keepdims=True)
    p1 = jnp.where(gcnt >= 2, g2, g1)

    s1f = flts[:, 16:48]
    s1s = flts[:, 48:80]
    s2f = flts[:, 80:112]
    s2s = flts[:, 112:144]
    v30f = flts[:, 144:176]
    v30s = flts[:, 176:208]
    v31f = flts[:, 208:240]
    v31s = flts[:, 240:272]

    own0 = first == g1

    def at_p0(q):
        return jnp.sum(jnp.where(own0, q, 0.0), axis=0, keepdims=True)

    def at_p1(qf, qs):
        return (jnp.sum(jnp.where(first == p1, qf, 0.0), axis=0, keepdims=True)
                + jnp.sum(jnp.where(second == p1, qs, 0.0), axis=0, keepdims=True))

    s1_i1 = jnp.where(f1sel > 0.5, at_p1(s1f, s1s), at_p0(s1f))
    s2_i2 = jnp.where(f2sel > 0.5, at_p1(s2f, s2s), at_p0(s2f))
    v30_i3 = jnp.where(f3sel > 0.5, at_p1(v30f, v30s), at_p0(v30f))
    v31_i3 = jnp.where(f3sel > 0.5, at_p1(v31f, v31s), at_p0(v31f))

    lsum = s1_i1 + s2_i2 - (_log1mexp(v30_i3) + _log1mexp(v31_i3))
    neg_total = jnp.sum(jnp.where(gcnt > 0, -lsum, 0.0))
    pos_total = jnp.sum(flts[:, 0:16])
    out_ref[...] = jnp.broadcast_to(neg_total - pos_total, (1, 1))


def kernel(volume1, volume2, volume3, xy_rel_id, yz_rel_id, xz_rel_id, flag):
    f32 = jnp.float32
    i32 = jnp.int32
    parts = [
        volume1.reshape(_NW, 2 * _CHUNK),
        volume2.reshape(_NW, 2 * _CHUNK),
        volume3.reshape(_NW, 2 * _CHUNK),
        lax.bitcast_convert_type(xy_rel_id.astype(i32).reshape(_NW, 2 * _CHUNK), f32),
        lax.bitcast_convert_type(yz_rel_id.astype(i32).reshape(_NW, 2 * _CHUNK), f32),
        lax.bitcast_convert_type(xz_rel_id.astype(i32).reshape(_NW, 2 * _CHUNK), f32),
        lax.bitcast_convert_type(flag.astype(i32).reshape(_NW, _CHUNK), f32),
    ]
    parts.append(jnp.zeros((_NW, _ROW - 13 * _CHUNK), jnp.float32))
    pk = jnp.concatenate(parts, axis=1).reshape(_NW * 64, 128)
    out_flt = _build_sc_scan()(pk)
    out = pl.pallas_call(
        _combine_body,
        out_shape=jax.ShapeDtypeStruct((1, 1), jnp.float32),
    )(out_flt.reshape(_NW, 23 * 16), jnp.asarray(_CONST_NP))
    return out[0, 0]


# speedup vs baseline: 2.6073x; 2.5960x over previous
import functools

import jax
import jax.numpy as jnp
import numpy as np
from jax import lax
from jax.experimental import pallas as pl
from jax.experimental.pallas import tpu as pltpu
from jax.experimental.pallas import tpu_sc as plsc

_B = 16384
_NW = 32
_CHUNK = 512
_WORDS = _CHUNK // 4
_R = 128
_C = 128
_BIG = 2**31 - 1

_POS = [(0, 4, 4), (0, 6, 4), (1, 5, 5), (1, 6, 5), (2, 4, 4), (2, 5, 5),
        (2, 6, 6), (2, 7, 7), (4, 0, 4), (4, 2, 4), (5, 1, 5), (5, 2, 5),
        (6, 2, 6), (7, 2, 7)]
_NEG = [(0, 4, 1), (0, 4, 2), (0, 6, 1), (0, 6, 2), (1, 5, 0), (1, 5, 2),
        (1, 6, 0), (1, 6, 2), (2, 4, 1), (2, 4, 2), (2, 5, 0), (2, 5, 2),
        (4, 0, 1), (4, 0, 2), (4, 2, 1), (4, 2, 2), (5, 1, 0), (5, 1, 2),
        (5, 2, 0), (5, 2, 2), (2, 7, 2), (7, 2, 2)]
_NR = len(_NEG)
assert _NR <= 32


def _dm(cat):
    return 0 if cat < 4 else 1


def _code(t):
    return t[0] * 64 + t[1] * 8 + t[2]


def _log1mexp(x):
    return jnp.log(1.0 - jnp.exp(x))


@functools.cache
def _build_sc_scan():
  mesh = plsc.VectorSubcoreMesh(core_axis_name="c", subcore_axis_name="s")

  @functools.partial(
    pl.kernel,
    mesh=mesh,
    compiler_params=pltpu.CompilerParams(needs_layout_passes=False),
    out_type=jax.ShapeDtypeStruct((_NW, 6, 16), jnp.int32),
    scratch_types=[
        pltpu.VMEM((_WORDS,), jnp.int32),
        pltpu.VMEM((_CHUNK,), jnp.int32),
        pltpu.VMEM((6, 16), jnp.int32),
        pltpu.SemaphoreType.DMA,
    ],
  )
  def _sc_scan(pk_hbm, out_hbm, pkc, codec, stg, sem):
    wid = lax.axis_index("c") * 16 + lax.axis_index("s")

    pltpu.async_copy(pk_hbm.at[wid], pkc, sem).wait()

    lane = lax.broadcasted_iota(jnp.int32, (16,), 0)

    def pa_body(i, carry):
        w = pkc[pl.ds(i * 16, 16)]
        for s in range(4):
            p = (w >> (8 * s)) & 0x7F
            x0 = p & 1
            x1 = (p >> 1) & 1
            y0 = (p >> 2) & 1
            y1 = (p >> 3) & 1
            z0 = (p >> 4) & 1
            z1 = (p >> 5) & 1
            fl4 = (p >> 6) * 4
            cx = 3 - 3 * x0 - 2 * x1 + 4 * x0 * x1 + fl4
            cy = 3 - 3 * y0 - 2 * y1 + 4 * y0 * y1 + fl4
            cz = 3 - 3 * z0 - 2 * z1 + 4 * z0 * z1 + fl4
            codec[pl.ds(i * 64 + s * 16, 16)] = cx * 64 + cy * 8 + cz
        return carry

    lax.fori_loop(0, _WORDS // 16, pa_body, 0)

    big16 = jnp.full((16,), _BIG, jnp.int32)
    zero16 = jnp.zeros((16,), jnp.int32)

    def scan_pair(ta, tb):
        def body(k, carry):
            m1a, m2a, ca, m1b, m2b, cb = carry
            code = codec[pl.ds(k * 16, 16)]
            idxv = (k >> 2) * 64 + (k & 3) + lane * 4
            sa = code == ta
            ca = ca + sa.astype(jnp.int32)
            mia = jnp.where(sa, idxv, _BIG)
            m2a = jnp.minimum(m2a, jnp.maximum(m1a, mia))
            m1a = jnp.minimum(m1a, mia)
            sb = code == tb
            cb = cb + sb.astype(jnp.int32)
            mib = jnp.where(sb, idxv, _BIG)
            m2b = jnp.minimum(m2b, jnp.maximum(m1b, mib))
            m1b = jnp.minimum(m1b, mib)
            return (m1a, m2a, ca, m1b, m2b, cb)

        init = (big16, big16, zero16, big16, big16, zero16)
        return lax.fori_loop(0, _CHUNK // 16, body, init)

    base = wid * _CHUNK
    cnt_v = [zero16, zero16]
    fg_v = [big16, big16]
    sg_v = [big16, big16]

    for r0 in range(0, _NR, 2):
        ta = _code(_NEG[r0])
        tb = _code(_NEG[r0 + 1]) if r0 + 1 < _NR else -1
        m1a, m2a, ca, m1b, m2b, cb = scan_pair(ta, tb)
        recs = [(r0, m1a, m2a, ca)]
        if r0 + 1 < _NR:
            recs.append((r0 + 1, m1b, m2b, cb))
        for r, m1, m2, cv in recs:
            first = jnp.min(m1)
            second = jnp.min(jnp.where(m1 == first, m2, m1))
            cnt = jnp.sum(cv)
            g = r // 16
            sel = lane == (r % 16)
            cnt_v[g] = jnp.where(sel, cnt, cnt_v[g])
            fg_v[g] = jnp.where(sel, jnp.where(first == _BIG, _BIG, first + base), fg_v[g])
            sg_v[g] = jnp.where(sel, jnp.where(second == _BIG, _BIG, second + base), sg_v[g])

    stg[0, :] = cnt_v[0]
    stg[1, :] = cnt_v[1]
    stg[2, :] = fg_v[0]
    stg[3, :] = fg_v[1]
    stg[4, :] = sg_v[0]
    stg[5, :] = sg_v[1]

    pltpu.sync_copy(stg, out_hbm.at[wid])

  return _sc_scan


def _combine_body(parts_ref, v10_r, v11_r, v20_r, v21_r, v30_r, v31_r,
                  x0_r, x1_r, y0_r, y1_r, z0_r, z1_r, fl_r, out_ref):
    v10, v11 = v10_r[...], v11_r[...]
    v20, v21 = v20_r[...], v21_r[...]
    v30, v31 = v30_r[...], v31_r[...]
    x0, x1 = x0_r[...], x1_r[...]
    y0, y1 = y0_r[...], y1_r[...]
    z0, z1 = z0_r[...], z1_r[...]
    fl = fl_r[...]

    four_fl = 4 * fl
    cx = 3 - 3 * x0 - 2 * x1 + 4 * x0 * x1 + four_fl
    cy = 3 - 3 * y0 - 2 * y1 + 4 * y0 * y1 + four_fl
    cz = 3 - 3 * z0 - 2 * z1 + 4 * z0 * z1 + four_fl
    code = cx * 64 + cy * 8 + cz

    idx = (lax.broadcasted_iota(jnp.int32, (_R, _C), 0) * _C
           + lax.broadcasted_iota(jnp.int32, (_R, _C), 1))

    v1c = (v10, v11)
    v2c = (v20, v21)
    v3c = (v30, v31)

    pos_acc = jnp.zeros((_R, _C), jnp.float32)
    for (xy, yz, xz) in _POS:
        t = _code((xy, yz, xz))
        w = v1c[_dm(xy)] + v2c[_dm(yz)] - v3c[_dm(xz)]
        pos_acc = pos_acc + jnp.where(code == t, w, 0.0)
    loss = -jnp.sum(pos_acc)

    parts = parts_ref[...]
    cnt = parts[:, 0:32]
    first = parts[:, 32:64]
    second = parts[:, 64:96]
    gcnt = jnp.sum(cnt, axis=0, keepdims=True)
    g1 = jnp.min(first, axis=0, keepdims=True)
    g2 = jnp.min(jnp.where(first == g1, second, first), axis=0, keepdims=True)
    p1 = jnp.where(gcnt >= 2, g2, g1)

    s1_full = v10 + v11
    s2_full = v20 + v21
    zero = jnp.zeros((_R, _C), jnp.float32)
    for r, (xy, yz, xz) in enumerate(_NEG):
        f1, f2, f3 = _dm(xy), _dm(yz), _dm(xz)
        cnt_r = gcnt[0, r]
        p0_r = jnp.minimum(g1[0, r], _B - 1)
        p1_r = jnp.minimum(p1[0, r], _B - 1)
        oh = (idx == p0_r, idx == p1_r)
        s12 = (jnp.sum(jnp.where(oh[f1], s1_full, zero))
               + jnp.sum(jnp.where(oh[f2], s2_full, zero)))
        v3a = jnp.sum(jnp.where(oh[f3], v30, zero))
        v3b = jnp.sum(jnp.where(oh[f3], v31, zero))
        lsum = s12 - (_log1mexp(v3a) + _log1mexp(v3b))
        loss = loss + jnp.where(cnt_r > 0, -lsum, 0.0)

    out_ref[...] = jnp.broadcast_to(loss, (1, 1))


def kernel(volume1, volume2, volume3, xy_rel_id, yz_rel_id, xz_rel_id, flag):
    i32 = jnp.int32
    xy = xy_rel_id.astype(i32)
    yz = yz_rel_id.astype(i32)
    xz = xz_rel_id.astype(i32)
    fl = flag.astype(i32)
    bits = (xy[:, 0] | (xy[:, 1] << 1) | (yz[:, 0] << 2) | (yz[:, 1] << 3)
            | (xz[:, 0] << 4) | (xz[:, 1] << 5) | (fl << 6))
    b4 = bits.reshape(-1, 4)
    packed = (b4[:, 0] | (b4[:, 1] << 8) | (b4[:, 2] << 16)
              | (b4[:, 3] << 24)).reshape(_NW, _WORDS)
    parts = _build_sc_scan()(packed)

    shp = (_R, _C)
    planes = (
        volume1[:, 0].reshape(shp), volume1[:, 1].reshape(shp),
        volume2[:, 0].reshape(shp), volume2[:, 1].reshape(shp),
        volume3[:, 0].reshape(shp), volume3[:, 1].reshape(shp),
        xy[:, 0].reshape(shp), xy[:, 1].reshape(shp),
        yz[:, 0].reshape(shp), yz[:, 1].reshape(shp),
        xz[:, 0].reshape(shp), xz[:, 1].reshape(shp),
        fl.reshape(shp),
    )
    out = pl.pallas_call(
        _combine_body,
        out_shape=jax.ShapeDtypeStruct((1, 1), jnp.float32),
    )(parts.reshape(_NW, 96), *planes)
    return out[0, 0]


# speedup vs baseline: 2.9858x; 1.1452x over previous
import functools

import jax
import jax.numpy as jnp
import numpy as np
from jax import lax
from jax.experimental import pallas as pl
from jax.experimental.pallas import tpu as pltpu
from jax.experimental.pallas import tpu_sc as plsc

_B = 16384
_NW = 32
_CHUNK = 512
_WORDS = _CHUNK // 4
_R = 128
_C = 128
_BIG = 2**31 - 1

_POS = [(0, 4, 4), (0, 6, 4), (1, 5, 5), (1, 6, 5), (2, 4, 4), (2, 5, 5),
        (2, 6, 6), (2, 7, 7), (4, 0, 4), (4, 2, 4), (5, 1, 5), (5, 2, 5),
        (6, 2, 6), (7, 2, 7)]
_NEG = [(0, 4, 1), (0, 4, 2), (0, 6, 1), (0, 6, 2), (1, 5, 0), (1, 5, 2),
        (1, 6, 0), (1, 6, 2), (2, 4, 1), (2, 4, 2), (2, 5, 0), (2, 5, 2),
        (4, 0, 1), (4, 0, 2), (4, 2, 1), (4, 2, 2), (5, 1, 0), (5, 1, 2),
        (5, 2, 0), (5, 2, 2), (2, 7, 2), (7, 2, 2)]
_NR = len(_NEG)
assert _NR <= 32


def _dm(cat):
    return 0 if cat < 4 else 1


def _code(t):
    return t[0] * 64 + t[1] * 8 + t[2]


def _log1mexp(x):
    return jnp.log(1.0 - jnp.exp(x))


@functools.cache
def _build_sc_scan():
  mesh = plsc.VectorSubcoreMesh(core_axis_name="c", subcore_axis_name="s")

  @functools.partial(
    pl.kernel,
    mesh=mesh,
    compiler_params=pltpu.CompilerParams(needs_layout_passes=False),
    out_type=jax.ShapeDtypeStruct((_NW, 6, 16), jnp.int32),
    scratch_types=[
        pltpu.VMEM((_WORDS,), jnp.int32),
        pltpu.VMEM((_CHUNK,), jnp.int32),
        pltpu.VMEM((6, 16), jnp.int32),
        pltpu.SemaphoreType.DMA,
    ],
  )
  def _sc_scan(pk_hbm, out_hbm, pkc, codec, stg, sem):
    wid = lax.axis_index("c") * 16 + lax.axis_index("s")

    pltpu.async_copy(pk_hbm.at[wid], pkc, sem).wait()

    lane = lax.broadcasted_iota(jnp.int32, (16,), 0)

    def pa_body(i, carry):
        w = pkc[pl.ds(i * 16, 16)]
        for s in range(4):
            p = (w >> (8 * s)) & 0x7F
            x0 = p & 1
            x1 = (p >> 1) & 1
            y0 = (p >> 2) & 1
            y1 = (p >> 3) & 1
            z0 = (p >> 4) & 1
            z1 = (p >> 5) & 1
            fl4 = (p >> 6) * 4
            cx = 3 - 3 * x0 - 2 * x1 + 4 * x0 * x1 + fl4
            cy = 3 - 3 * y0 - 2 * y1 + 4 * y0 * y1 + fl4
            cz = 3 - 3 * z0 - 2 * z1 + 4 * z0 * z1 + fl4
            codec[pl.ds(i * 64 + s * 16, 16)] = cx * 64 + cy * 8 + cz
        return carry

    lax.fori_loop(0, _WORDS // 16, pa_body, 0)

    big16 = jnp.full((16,), _BIG, jnp.int32)
    zero16 = jnp.zeros((16,), jnp.int32)

    def scan_pair(ta, tb):
        def body(k, carry):
            m1a, m2a, ca, m1b, m2b, cb = carry
            code = codec[pl.ds(k * 16, 16)]
            idxv = (k >> 2) * 64 + (k & 3) + lane * 4
            sa = code == ta
            ca = ca + sa.astype(jnp.int32)
            mia = jnp.where(sa, idxv, _BIG)
            m2a = jnp.minimum(m2a, jnp.maximum(m1a, mia))
            m1a = jnp.minimum(m1a, mia)
            sb = code == tb
            cb = cb + sb.astype(jnp.int32)
            mib = jnp.where(sb, idxv, _BIG)
            m2b = jnp.minimum(m2b, jnp.maximum(m1b, mib))
            m1b = jnp.minimum(m1b, mib)
            return (m1a, m2a, ca, m1b, m2b, cb)

        init = (big16, big16, zero16, big16, big16, zero16)
        return lax.fori_loop(0, _CHUNK // 16, body, init)

    base = wid * _CHUNK
    cnt_v = [zero16, zero16]
    fg_v = [big16, big16]
    sg_v = [big16, big16]

    for r0 in range(0, _NR, 2):
        ta = _code(_NEG[r0])
        tb = _code(_NEG[r0 + 1]) if r0 + 1 < _NR else -1
        m1a, m2a, ca, m1b, m2b, cb = scan_pair(ta, tb)
        recs = [(r0, m1a, m2a, ca)]
        if r0 + 1 < _NR:
            recs.append((r0 + 1, m1b, m2b, cb))
        for r, m1, m2, cv in recs:
            first = jnp.min(m1)
            second = jnp.min(jnp.where(m1 == first, m2, m1))
            cnt = jnp.sum(cv)
            g = r // 16
            sel = lane == (r % 16)
            cnt_v[g] = jnp.where(sel, cnt, cnt_v[g])
            fg_v[g] = jnp.where(sel, jnp.where(first == _BIG, _BIG, first + base), fg_v[g])
            sg_v[g] = jnp.where(sel, jnp.where(second == _BIG, _BIG, second + base), sg_v[g])

    stg[0, :] = cnt_v[0]
    stg[1, :] = cnt_v[1]
    stg[2, :] = fg_v[0]
    stg[3, :] = fg_v[1]
    stg[4, :] = sg_v[0]
    stg[5, :] = sg_v[1]

    pltpu.sync_copy(stg, out_hbm.at[wid])

  return _sc_scan


def _combine_body(parts_ref, v10_r, v11_r, v20_r, v21_r, v30_r, v31_r,
                  x0_r, x1_r, y0_r, y1_r, z0_r, z1_r, fl_r, out_ref):
    v10, v11 = v10_r[...], v11_r[...]
    v20, v21 = v20_r[...], v21_r[...]
    v30, v31 = v30_r[...], v31_r[...]
    x0, x1 = x0_r[...], x1_r[...]
    y0, y1 = y0_r[...], y1_r[...]
    z0, z1 = z0_r[...], z1_r[...]
    fl = fl_r[...]

    four_fl = 4 * fl
    cx = 3 - 3 * x0 - 2 * x1 + 4 * x0 * x1 + four_fl
    cy = 3 - 3 * y0 - 2 * y1 + 4 * y0 * y1 + four_fl
    cz = 3 - 3 * z0 - 2 * z1 + 4 * z0 * z1 + four_fl
    code = cx * 64 + cy * 8 + cz

    idx = (lax.broadcasted_iota(jnp.int32, (_R, _C), 0) * _C
           + lax.broadcasted_iota(jnp.int32, (_R, _C), 1))

    v1c = (v10, v11)
    v2c = (v20, v21)
    v3c = (v30, v31)

    pos_acc = jnp.zeros((_R, _C), jnp.float32)
    for (xy, yz, xz) in _POS:
        t = _code((xy, yz, xz))
        w = v1c[_dm(xy)] + v2c[_dm(yz)] - v3c[_dm(xz)]
        pos_acc = pos_acc + jnp.where(code == t, w, 0.0)
    loss = -jnp.sum(pos_acc)

    parts = parts_ref[...]
    cnt = parts[:, 0:32]
    first = parts[:, 32:64]
    second = parts[:, 64:96]
    gcnt = jnp.sum(cnt, axis=0, keepdims=True)
    g1 = jnp.min(first, axis=0, keepdims=True)
    g2 = jnp.min(jnp.where(first == g1, second, first), axis=0, keepdims=True)
    p1 = jnp.where(gcnt >= 2, g2, g1)

    s1_full = v10 + v11
    s2_full = v20 + v21
    l3_full = _log1mexp(v30) + _log1mexp(v31)

    false_p = jnp.zeros((_R, _C), jnp.bool_)
    m1p, m2p, m3p = false_p, false_p, false_p
    for r, (xy, yz, xz) in enumerate(_NEG):
        f1, f2, f3 = _dm(xy), _dm(yz), _dm(xz)
        live = gcnt[0, r] > 0
        p0_r = jnp.minimum(g1[0, r], _B - 1)
        p1_r = jnp.minimum(p1[0, r], _B - 1)
        oh = ((idx == p0_r) & live, (idx == p1_r) & live)
        m1p = m1p | oh[f1]
        m2p = m2p | oh[f2]
        m3p = m3p | oh[f3]
    zero = jnp.zeros((_R, _C), jnp.float32)
    loss = (loss - jnp.sum(jnp.where(m1p, s1_full, zero))
            - jnp.sum(jnp.where(m2p, s2_full, zero))
            + jnp.sum(jnp.where(m3p, l3_full, zero)))

    out_ref[...] = jnp.broadcast_to(loss, (1, 1))


def kernel(volume1, volume2, volume3, xy_rel_id, yz_rel_id, xz_rel_id, flag):
    i32 = jnp.int32
    xy = xy_rel_id.astype(i32)
    yz = yz_rel_id.astype(i32)
    xz = xz_rel_id.astype(i32)
    fl = flag.astype(i32)
    bits = (xy[:, 0] | (xy[:, 1] << 1) | (yz[:, 0] << 2) | (yz[:, 1] << 3)
            | (xz[:, 0] << 4) | (xz[:, 1] << 5) | (fl << 6))
    b4 = bits.reshape(-1, 4)
    packed = (b4[:, 0] | (b4[:, 1] << 8) | (b4[:, 2] << 16)
              | (b4[:, 3] << 24)).reshape(_NW, _WORDS)
    parts = _build_sc_scan()(packed)

    shp = (_R, _C)
    planes = (
        volume1[:, 0].reshape(shp), volume1[:, 1].reshape(shp),
        volume2[:, 0].reshape(shp), volume2[:, 1].reshape(shp),
        volume3[:, 0].reshape(shp), volume3[:, 1].reshape(shp),
        xy[:, 0].reshape(shp), xy[:, 1].reshape(shp),
        yz[:, 0].reshape(shp), yz[:, 1].reshape(shp),
        xz[:, 0].reshape(shp), xz[:, 1].reshape(shp),
        fl.reshape(shp),
    )
    out = pl.pallas_call(
        _combine_body,
        out_shape=jax.ShapeDtypeStruct((1, 1), jnp.float32),
    )(parts.reshape(_NW, 96), *planes)
    return out[0, 0]
